# Initial kernel scaffold; baseline (speedup 1.0000x reference)
import jax
import jax.numpy as jnp
from jax.experimental import pallas as pl


def kernel(xyz1, xyz2, points1, points2, W0, b0, g0, beta0, W1, b1, g1, beta1):
    raise NotImplementedError("write your pallas kernel here")


# speedup vs baseline: 17.7502x; 17.7502x over previous
import functools

import jax
import jax.numpy as jnp
from jax import lax
from jax.experimental import pallas as pl
from jax.experimental.pallas import tpu as pltpu
from jax.experimental.pallas import tpu_sc as plsc

B, N, S, D1, D2 = 16, 4096, 1024, 64, 128
MLP0, MLP1 = 128, 128
R = B * N

NC, NS, LANES = 2, 16, 16
NW = NC * NS
RPW = R // NW
CHUNK = 128
SUB = 16

NB1 = 512
NB3 = 512
NB4 = 512
NB5 = 2048


def _nn3_body(x1t_ref, xyz2_ref, i1, i2, i3, w1, w2, w3):
    b = pl.program_id(0)
    x1 = x1t_ref[0]
    x2 = xyz2_ref[0]
    n1 = jnp.sum(x1 * x1, axis=1, keepdims=True)
    n2 = jnp.sum(x2 * x2, axis=0, keepdims=True)
    ab = lax.dot_general(x1, x2, (((1,), (0,)), ((), ())),
                         preferred_element_type=jnp.float32)
    d = -2.0 * ab
    d = d + n1
    d = d + n2

    iota = lax.broadcasted_iota(jnp.int32, (NB1, S), 1)
    ms, js = [], []
    for k in range(3):
        m = jnp.min(d, axis=1, keepdims=True)
        cand = jnp.where(d == m, iota, S)
        j = jnp.min(cand, axis=1, keepdims=True)
        ms.append(m)
        js.append(j)
        if k < 2:
            d = jnp.where(cand == j, jnp.float32(jnp.inf), d)

    r1 = 1.0 / (ms[0] + 1e-8)
    r2 = 1.0 / (ms[1] + 1e-8)
    r3 = 1.0 / (ms[2] + 1e-8)
    norm = r1 + r2 + r3
    w1[0] = r1 / norm
    w2[0] = r2 / norm
    w3[0] = r3 / norm
    base = b * S
    i1[0] = js[0] + base
    i2[0] = js[1] + base
    i3[0] = js[2] + base


def _nn3(x1t, xyz2):
    grid = (B, N // NB1)
    out = pl.pallas_call(
        _nn3_body,
        grid=grid,
        in_specs=[
            pl.BlockSpec((1, NB1, 3), lambda b, n: (b, n, 0)),
            pl.BlockSpec((1, 3, S), lambda b, n: (b, 0, 0)),
        ],
        out_specs=[pl.BlockSpec((1, NB1, 1), lambda b, n: (b, n, 0))] * 6,
        out_shape=[jax.ShapeDtypeStruct((B, N, 1), jnp.int32)] * 3
        + [jax.ShapeDtypeStruct((B, N, 1), jnp.float32)] * 3,
    )(x1t, xyz2)
    return out


def _sc_interp_body(p2_hbm, i1_hbm, i2_hbm, i3_hbm, w1_hbm, w2_hbm, w3_hbm,
                    out_hbm, i1_v, i2_v, i3_v, w1_v, w2_v, w3_v,
                    r1_v, r2_v, r3_v, o_v, sem):
    wid = lax.axis_index("s") * NC + lax.axis_index("c")
    base = wid * RPW
    pltpu.sync_copy(i1_hbm.at[pl.ds(base, RPW)], i1_v)
    pltpu.sync_copy(i2_hbm.at[pl.ds(base, RPW)], i2_v)
    pltpu.sync_copy(i3_hbm.at[pl.ds(base, RPW)], i3_v)
    pltpu.sync_copy(w1_hbm.at[pl.ds(base, RPW)], w1_v)
    pltpu.sync_copy(w2_hbm.at[pl.ds(base, RPW)], w2_v)
    pltpu.sync_copy(w3_hbm.at[pl.ds(base, RPW)], w3_v)

    def round_body(g, _):
        off = g * CHUNK
        cp1 = pltpu.async_copy(p2_hbm.at[i1_v.at[pl.ds(off, CHUNK)]], r1_v, sem)
        cp2 = pltpu.async_copy(p2_hbm.at[i2_v.at[pl.ds(off, CHUNK)]], r2_v, sem)
        cp3 = pltpu.async_copy(p2_hbm.at[i3_v.at[pl.ds(off, CHUNK)]], r3_v, sem)
        cp1.wait()
        cp2.wait()
        cp3.wait()

        def sub_body(t, _):
            r0 = t * SUB
            wv1 = w1_v[pl.ds(off + r0, SUB)]
            wv2 = w2_v[pl.ds(off + r0, SUB)]
            wv3 = w3_v[pl.ds(off + r0, SUB)]
            for r in range(SUB):
                ids = jnp.full((LANES,), r, jnp.int32)
                a1 = wv1.at[ids].get(mode="promise_in_bounds")
                a2 = wv2.at[ids].get(mode="promise_in_bounds")
                a3 = wv3.at[ids].get(mode="promise_in_bounds")
                row = r0 + r
                for c in range(D2 // LANES):
                    sl = pl.ds(c * LANES, LANES)
                    o_v[row, sl] = (a1 * r1_v[row, sl]
                                    + a2 * r2_v[row, sl]
                                    + a3 * r3_v[row, sl])
            return _

        lax.fori_loop(0, CHUNK // SUB, sub_body, 0, unroll=False)
        pltpu.sync_copy(o_v, out_hbm.at[pl.ds(base + off, CHUNK)])
        return _

    lax.fori_loop(0, RPW // CHUNK, round_body, 0, unroll=False)


def _sc_interp(p2_flat, i1, i2, i3, w1, w2, w3):
    mesh = plsc.VectorSubcoreMesh(core_axis_name="c", subcore_axis_name="s")
    fn = functools.partial(
        pl.kernel,
        out_type=jax.ShapeDtypeStruct((R, D2), jnp.float32),
        mesh=mesh,
        scratch_types=[
            pltpu.VMEM((RPW,), jnp.int32),
            pltpu.VMEM((RPW,), jnp.int32),
            pltpu.VMEM((RPW,), jnp.int32),
            pltpu.VMEM((RPW,), jnp.float32),
            pltpu.VMEM((RPW,), jnp.float32),
            pltpu.VMEM((RPW,), jnp.float32),
            pltpu.VMEM((CHUNK, D2), jnp.float32),
            pltpu.VMEM((CHUNK, D2), jnp.float32),
            pltpu.VMEM((CHUNK, D2), jnp.float32),
            pltpu.VMEM((CHUNK, D2), jnp.float32),
            pltpu.SemaphoreType.DMA,
        ],
    )(_sc_interp_body)
    return fn(p2_flat, i1, i2, i3, w1, w2, w3)


def _conv0_body(p1_ref, it_ref, w0_ref, b0_ref, y_ref, st_ref):
    first = (pl.program_id(0) == 0) & (pl.program_id(1) == 0)

    @pl.when(first)
    def _():
        st_ref[...] = jnp.zeros_like(st_ref)

    w0a = w0_ref[:, :D1]
    w0b = w0_ref[:, D1:]
    t1 = lax.dot_general(w0a, p1_ref[0], (((1,), (0,)), ((), ())),
                         preferred_element_type=jnp.float32)
    t2 = lax.dot_general(w0b, it_ref[0], (((1,), (1,)), ((), ())),
                         preferred_element_type=jnp.float32)
    y = t1 + t2 + b0_ref[...]
    y_ref[0] = y
    st_ref[:, 0:1] += jnp.sum(y, axis=1, keepdims=True)
    st_ref[:, 1:2] += jnp.sum(y * y, axis=1, keepdims=True)


def _conv0(points1, interp3d, W0, b0c):
    grid = (B, N // NB3)
    return pl.pallas_call(
        _conv0_body,
        grid=grid,
        in_specs=[
            pl.BlockSpec((1, D1, NB3), lambda b, n: (b, 0, n)),
            pl.BlockSpec((1, NB3, D2), lambda b, n: (b, n, 0)),
            pl.BlockSpec((MLP0, D1 + D2), lambda b, n: (0, 0)),
            pl.BlockSpec((MLP0, 1), lambda b, n: (0, 0)),
        ],
        out_specs=[
            pl.BlockSpec((1, MLP0, NB3), lambda b, n: (b, 0, n)),
            pl.BlockSpec((MLP0, 128), lambda b, n: (0, 0)),
        ],
        out_shape=[
            jax.ShapeDtypeStruct((B, MLP0, N), jnp.float32),
            jax.ShapeDtypeStruct((MLP0, 128), jnp.float32),
        ],
    )(points1, interp3d, W0, b0c)


def _conv1_body(y0_ref, st0_ref, g0_ref, be0_ref, w1_ref, b1_ref,
                y_ref, st_ref):
    first = (pl.program_id(0) == 0) & (pl.program_id(1) == 0)

    @pl.when(first)
    def _():
        st_ref[...] = jnp.zeros_like(st_ref)

    cnt = jnp.float32(B * N)
    mean = st0_ref[:, 0:1] / cnt
    var = st0_ref[:, 1:2] / cnt - mean * mean
    inv = lax.rsqrt(var + 1e-5)
    scale = g0_ref[...] * inv
    shift = be0_ref[...] - mean * scale
    h = jnp.maximum(y0_ref[0] * scale + shift, 0.0)
    y = lax.dot_general(w1_ref[...], h, (((1,), (0,)), ((), ())),
                        preferred_element_type=jnp.float32) + b1_ref[...]
    y_ref[0] = y
    st_ref[:, 0:1] += jnp.sum(y, axis=1, keepdims=True)
    st_ref[:, 1:2] += jnp.sum(y * y, axis=1, keepdims=True)


def _conv1(y0, st0, g0c, beta0c, W1, b1c):
    grid = (B, N // NB4)
    return pl.pallas_call(
        _conv1_body,
        grid=grid,
        in_specs=[
            pl.BlockSpec((1, MLP0, NB4), lambda b, n: (b, 0, n)),
            pl.BlockSpec((MLP0, 128), lambda b, n: (0, 0)),
            pl.BlockSpec((MLP0, 1), lambda b, n: (0, 0)),
            pl.BlockSpec((MLP0, 1), lambda b, n: (0, 0)),
            pl.BlockSpec((MLP1, MLP0), lambda b, n: (0, 0)),
            pl.BlockSpec((MLP1, 1), lambda b, n: (0, 0)),
        ],
        out_specs=[
            pl.BlockSpec((1, MLP1, NB4), lambda b, n: (b, 0, n)),
            pl.BlockSpec((MLP1, 128), lambda b, n: (0, 0)),
        ],
        out_shape=[
            jax.ShapeDtypeStruct((B, MLP1, N), jnp.float32),
            jax.ShapeDtypeStruct((MLP1, 128), jnp.float32),
        ],
    )(y0, st0, g0c, beta0c, W1, b1c)


def _bn1_body(y1_ref, st1_ref, g1_ref, be1_ref, out_ref):
    cnt = jnp.float32(B * N)
    mean = st1_ref[:, 0:1] / cnt
    var = st1_ref[:, 1:2] / cnt - mean * mean
    inv = lax.rsqrt(var + 1e-5)
    scale = g1_ref[...] * inv
    shift = be1_ref[...] - mean * scale
    out_ref[0] = jnp.maximum(y1_ref[0] * scale + shift, 0.0)


def _bn1(y1, st1, g1c, beta1c):
    grid = (B, N // NB5)
    return pl.pallas_call(
        _bn1_body,
        grid=grid,
        in_specs=[
            pl.BlockSpec((1, MLP1, NB5), lambda b, n: (b, 0, n)),
            pl.BlockSpec((MLP1, 128), lambda b, n: (0, 0)),
            pl.BlockSpec((MLP1, 1), lambda b, n: (0, 0)),
            pl.BlockSpec((MLP1, 1), lambda b, n: (0, 0)),
        ],
        out_specs=pl.BlockSpec((1, MLP1, NB5), lambda b, n: (b, 0, n)),
        out_shape=jax.ShapeDtypeStruct((B, MLP1, N), jnp.float32),
    )(y1, st1, g1c, beta1c)


@jax.jit
def kernel(xyz1, xyz2, points1, points2, W0, b0, g0, beta0, W1, b1, g1, beta1):
    i1, i2, i3, w1, w2, w3 = _nn3(jnp.transpose(xyz1, (0, 2, 1)), xyz2)
    p2_flat = jnp.transpose(points2, (0, 2, 1)).reshape(B * S, D2)
    interp = _sc_interp(
        p2_flat,
        i1.reshape(R), i2.reshape(R), i3.reshape(R),
        w1.reshape(R), w2.reshape(R), w3.reshape(R),
    )
    interp3d = interp.reshape(B, N, D2)
    y0, st0 = _conv0(points1, interp3d, W0, b0.reshape(MLP0, 1))
    y1, st1 = _conv1(y0, st0, g0.reshape(MLP0, 1), beta0.reshape(MLP0, 1),
                     W1, b1.reshape(MLP1, 1))
    return _bn1(y1, st1, g1.reshape(MLP1, 1), beta1.reshape(MLP1, 1))


# speedup vs baseline: 23.4873x; 1.3232x over previous
import functools

import jax
import jax.numpy as jnp
from jax import lax
from jax.experimental import pallas as pl
from jax.experimental.pallas import tpu as pltpu
from jax.experimental.pallas import tpu_sc as plsc

B, N, S, D1, D2 = 16, 4096, 1024, 64, 128
MLP0, MLP1 = 128, 128
R = B * N

NC, NS, LANES = 2, 16, 16
NW = NC * NS
RPW = R // NW
CHUNK = 128
SUB = 16

NB1 = 512
NB3 = 1024
NB4 = 1024
NB5 = 2048


def _nn3_body(xyz1_ref, x2t_ref, i1, i2, i3, w1, w2, w3):
    b = pl.program_id(0)
    x1 = xyz1_ref[0]
    x2 = x2t_ref[0]
    n1 = (x1[0:1, :] * x1[0:1, :] + x1[1:2, :] * x1[1:2, :]) \
        + x1[2:3, :] * x1[2:3, :]
    n2 = (x2[:, 0:1] * x2[:, 0:1] + x2[:, 1:2] * x2[:, 1:2]) \
        + x2[:, 2:3] * x2[:, 2:3]
    ab = lax.dot_general(x2, x1, (((1,), (0,)), ((), ())),
                         preferred_element_type=jnp.float32)
    d = -2.0 * ab
    d = d + n1
    d = d + n2

    iota = lax.broadcasted_iota(jnp.int32, (S, NB1), 0)
    ms, js = [], []
    for k in range(3):
        m = jnp.min(d, axis=0, keepdims=True)
        eq = d == m
        cand = jnp.where(eq, iota, S)
        j = jnp.min(cand, axis=0, keepdims=True)
        ms.append(m)
        js.append(j)
        if k < 2:
            d = jnp.where(cand == j, jnp.float32(jnp.inf), d)

    r1 = 1.0 / (ms[0] + 1e-8)
    r2 = 1.0 / (ms[1] + 1e-8)
    r3 = 1.0 / (ms[2] + 1e-8)
    norm = r1 + r2 + r3
    w1[0] = r1 / norm
    w2[0] = r2 / norm
    w3[0] = r3 / norm
    base = b * S
    i1[0] = js[0] + base
    i2[0] = js[1] + base
    i3[0] = js[2] + base


def _nn3(xyz1, x2t):
    grid = (B, N // NB1)
    out = pl.pallas_call(
        _nn3_body,
        grid=grid,
        in_specs=[
            pl.BlockSpec((1, 3, NB1), lambda b, n: (b, 0, n)),
            pl.BlockSpec((1, S, 3), lambda b, n: (b, 0, 0)),
        ],
        out_specs=[pl.BlockSpec((1, 1, NB1), lambda b, n: (b, 0, n))] * 6,
        out_shape=[jax.ShapeDtypeStruct((B, 1, N), jnp.int32)] * 3
        + [jax.ShapeDtypeStruct((B, 1, N), jnp.float32)] * 3,
    )(xyz1, x2t)
    return out


def _sc_interp_body(p2_hbm, i1_hbm, i2_hbm, i3_hbm, w1_hbm, w2_hbm, w3_hbm,
                    out_hbm, i1_v, i2_v, i3_v, w1_v, w2_v, w3_v,
                    r1_v, r2_v, r3_v, o_v, sem):
    wid = lax.axis_index("s") * NC + lax.axis_index("c")
    base = wid * RPW
    pltpu.sync_copy(i1_hbm.at[pl.ds(base, RPW)], i1_v)
    pltpu.sync_copy(i2_hbm.at[pl.ds(base, RPW)], i2_v)
    pltpu.sync_copy(i3_hbm.at[pl.ds(base, RPW)], i3_v)
    pltpu.sync_copy(w1_hbm.at[pl.ds(base, RPW)], w1_v)
    pltpu.sync_copy(w2_hbm.at[pl.ds(base, RPW)], w2_v)
    pltpu.sync_copy(w3_hbm.at[pl.ds(base, RPW)], w3_v)

    def round_body(g, _):
        off = g * CHUNK
        cp1 = pltpu.async_copy(p2_hbm.at[i1_v.at[pl.ds(off, CHUNK)]], r1_v, sem)
        cp2 = pltpu.async_copy(p2_hbm.at[i2_v.at[pl.ds(off, CHUNK)]], r2_v, sem)
        cp3 = pltpu.async_copy(p2_hbm.at[i3_v.at[pl.ds(off, CHUNK)]], r3_v, sem)
        cp1.wait()
        cp2.wait()
        cp3.wait()

        def sub_body(t, _):
            r0 = t * SUB
            wv1 = w1_v[pl.ds(off + r0, SUB)]
            wv2 = w2_v[pl.ds(off + r0, SUB)]
            wv3 = w3_v[pl.ds(off + r0, SUB)]
            for r in range(SUB):
                ids = jnp.full((LANES,), r, jnp.int32)
                a1 = wv1.at[ids].get(mode="promise_in_bounds")
                a2 = wv2.at[ids].get(mode="promise_in_bounds")
                a3 = wv3.at[ids].get(mode="promise_in_bounds")
                row = r0 + r
                for c in range(D2 // LANES):
                    sl = pl.ds(c * LANES, LANES)
                    o_v[row, sl] = (a1 * r1_v[row, sl]
                                    + a2 * r2_v[row, sl]
                                    + a3 * r3_v[row, sl])
            return _

        lax.fori_loop(0, CHUNK // SUB, sub_body, 0, unroll=False)
        pltpu.sync_copy(o_v, out_hbm.at[pl.ds(base + off, CHUNK)])
        return _

    lax.fori_loop(0, RPW // CHUNK, round_body, 0, unroll=False)


def _sc_interp(p2_flat, i1, i2, i3, w1, w2, w3):
    mesh = plsc.VectorSubcoreMesh(core_axis_name="c", subcore_axis_name="s")
    fn = functools.partial(
        pl.kernel,
        out_type=jax.ShapeDtypeStruct((R, D2), jnp.float32),
        mesh=mesh,
        scratch_types=[
            pltpu.VMEM((RPW,), jnp.int32),
            pltpu.VMEM((RPW,), jnp.int32),
            pltpu.VMEM((RPW,), jnp.int32),
            pltpu.VMEM((RPW,), jnp.float32),
            pltpu.VMEM((RPW,), jnp.float32),
            pltpu.VMEM((RPW,), jnp.float32),
            pltpu.VMEM((CHUNK, D2), jnp.float32),
            pltpu.VMEM((CHUNK, D2), jnp.float32),
            pltpu.VMEM((CHUNK, D2), jnp.float32),
            pltpu.VMEM((CHUNK, D2), jnp.float32),
            pltpu.SemaphoreType.DMA,
        ],
    )(_sc_interp_body)
    return fn(p2_flat, i1, i2, i3, w1, w2, w3)


def _conv0_body(p1_ref, it_ref, w0_ref, b0_ref, y_ref, st_ref):
    first = (pl.program_id(0) == 0) & (pl.program_id(1) == 0)

    @pl.when(first)
    def _():
        st_ref[...] = jnp.zeros_like(st_ref)

    w0a = w0_ref[:, :D1]
    w0b = w0_ref[:, D1:]
    t1 = lax.dot_general(w0a, p1_ref[0], (((1,), (0,)), ((), ())),
                         preferred_element_type=jnp.float32)
    t2 = lax.dot_general(w0b, it_ref[0], (((1,), (1,)), ((), ())),
                         preferred_element_type=jnp.float32)
    y = t1 + t2 + b0_ref[...]
    y_ref[0] = y
    st_ref[:, 0:1] += jnp.sum(y, axis=1, keepdims=True)
    st_ref[:, 1:2] += jnp.sum(y * y, axis=1, keepdims=True)


def _conv0(points1, interp3d, W0, b0c):
    grid = (B, N // NB3)
    return pl.pallas_call(
        _conv0_body,
        grid=grid,
        in_specs=[
            pl.BlockSpec((1, D1, NB3), lambda b, n: (b, 0, n)),
            pl.BlockSpec((1, NB3, D2), lambda b, n: (b, n, 0)),
            pl.BlockSpec((MLP0, D1 + D2), lambda b, n: (0, 0)),
            pl.BlockSpec((MLP0, 1), lambda b, n: (0, 0)),
        ],
        out_specs=[
            pl.BlockSpec((1, MLP0, NB3), lambda b, n: (b, 0, n)),
            pl.BlockSpec((MLP0, 128), lambda b, n: (0, 0)),
        ],
        out_shape=[
            jax.ShapeDtypeStruct((B, MLP0, N), jnp.float32),
            jax.ShapeDtypeStruct((MLP0, 128), jnp.float32),
        ],
    )(points1, interp3d, W0, b0c)


def _conv1_body(y0_ref, st0_ref, g0_ref, be0_ref, w1_ref, b1_ref,
                y_ref, st_ref):
    first = (pl.program_id(0) == 0) & (pl.program_id(1) == 0)

    @pl.when(first)
    def _():
        st_ref[...] = jnp.zeros_like(st_ref)

    cnt = jnp.float32(B * N)
    mean = st0_ref[:, 0:1] / cnt
    var = st0_ref[:, 1:2] / cnt - mean * mean
    inv = lax.rsqrt(var + 1e-5)
    scale = g0_ref[...] * inv
    shift = be0_ref[...] - mean * scale
    h = jnp.maximum(y0_ref[0] * scale + shift, 0.0)
    y = lax.dot_general(w1_ref[...], h, (((1,), (0,)), ((), ())),
                        preferred_element_type=jnp.float32) + b1_ref[...]
    y_ref[0] = y
    st_ref[:, 0:1] += jnp.sum(y, axis=1, keepdims=True)
    st_ref[:, 1:2] += jnp.sum(y * y, axis=1, keepdims=True)


def _conv1(y0, st0, g0c, beta0c, W1, b1c):
    grid = (B, N // NB4)
    return pl.pallas_call(
        _conv1_body,
        grid=grid,
        in_specs=[
            pl.BlockSpec((1, MLP0, NB4), lambda b, n: (b, 0, n)),
            pl.BlockSpec((MLP0, 128), lambda b, n: (0, 0)),
            pl.BlockSpec((MLP0, 1), lambda b, n: (0, 0)),
            pl.BlockSpec((MLP0, 1), lambda b, n: (0, 0)),
            pl.BlockSpec((MLP1, MLP0), lambda b, n: (0, 0)),
            pl.BlockSpec((MLP1, 1), lambda b, n: (0, 0)),
        ],
        out_specs=[
            pl.BlockSpec((1, MLP1, NB4), lambda b, n: (b, 0, n)),
            pl.BlockSpec((MLP1, 128), lambda b, n: (0, 0)),
        ],
        out_shape=[
            jax.ShapeDtypeStruct((B, MLP1, N), jnp.float32),
            jax.ShapeDtypeStruct((MLP1, 128), jnp.float32),
        ],
    )(y0, st0, g0c, beta0c, W1, b1c)


def _bn1_body(y1_ref, st1_ref, g1_ref, be1_ref, out_ref):
    cnt = jnp.float32(B * N)
    mean = st1_ref[:, 0:1] / cnt
    var = st1_ref[:, 1:2] / cnt - mean * mean
    inv = lax.rsqrt(var + 1e-5)
    scale = g1_ref[...] * inv
    shift = be1_ref[...] - mean * scale
    out_ref[0] = jnp.maximum(y1_ref[0] * scale + shift, 0.0)


def _bn1(y1, st1, g1c, beta1c):
    grid = (B, N // NB5)
    return pl.pallas_call(
        _bn1_body,
        grid=grid,
        in_specs=[
            pl.BlockSpec((1, MLP1, NB5), lambda b, n: (b, 0, n)),
            pl.BlockSpec((MLP1, 128), lambda b, n: (0, 0)),
            pl.BlockSpec((MLP1, 1), lambda b, n: (0, 0)),
            pl.BlockSpec((MLP1, 1), lambda b, n: (0, 0)),
        ],
        out_specs=pl.BlockSpec((1, MLP1, NB5), lambda b, n: (b, 0, n)),
        out_shape=jax.ShapeDtypeStruct((B, MLP1, N), jnp.float32),
    )(y1, st1, g1c, beta1c)


@jax.jit
def kernel(xyz1, xyz2, points1, points2, W0, b0, g0, beta0, W1, b1, g1, beta1):
    i1, i2, i3, w1, w2, w3 = _nn3(xyz1, jnp.transpose(xyz2, (0, 2, 1)))
    p2_flat = jnp.transpose(points2, (0, 2, 1)).reshape(B * S, D2)
    interp = _sc_interp(
        p2_flat,
        i1.reshape(R), i2.reshape(R), i3.reshape(R),
        w1.reshape(R), w2.reshape(R), w3.reshape(R),
    )
    interp3d = interp.reshape(B, N, D2)
    y0, st0 = _conv0(points1, interp3d, W0, b0.reshape(MLP0, 1))
    y1, st1 = _conv1(y0, st0, g0.reshape(MLP0, 1), beta0.reshape(MLP0, 1),
                     W1, b1.reshape(MLP1, 1))
    return _bn1(y1, st1, g1.reshape(MLP1, 1), beta1.reshape(MLP1, 1))


# speedup vs baseline: 24.7991x; 1.0559x over previous
import functools

import jax
import jax.numpy as jnp
from jax import lax
from jax.experimental import pallas as pl
from jax.experimental.pallas import tpu as pltpu
from jax.experimental.pallas import tpu_sc as plsc

B, N, S, D1, D2 = 16, 4096, 1024, 64, 128
MLP0, MLP1 = 128, 128
R = B * N

NC, NS, LANES = 2, 16, 16
NW = NC * NS
RPW = R // NW
CHUNK = 128
SUB = 16

NB1 = 1024
NB3 = 1024
NB4 = 1024
NB5 = 2048


def _nn3_body(xyz1_ref, x2t_ref, i1, i2, i3, w1, w2, w3):
    b = pl.program_id(0)
    x1 = xyz1_ref[0]
    x2 = x2t_ref[0]
    n1 = (x1[0:1, :] * x1[0:1, :] + x1[1:2, :] * x1[1:2, :]) \
        + x1[2:3, :] * x1[2:3, :]
    n2 = (x2[:, 0:1] * x2[:, 0:1] + x2[:, 1:2] * x2[:, 1:2]) \
        + x2[:, 2:3] * x2[:, 2:3]
    ab = lax.dot_general(x2, x1, (((1,), (0,)), ((), ())),
                         preferred_element_type=jnp.float32)
    d = -2.0 * ab
    d = d + n1
    d = d + n2

    iota = lax.broadcasted_iota(jnp.int32, (S, NB1), 0)
    ms, js = [], []
    for k in range(3):
        m = jnp.min(d, axis=0, keepdims=True)
        eq = d == m
        cand = jnp.where(eq, iota, S)
        j = jnp.min(cand, axis=0, keepdims=True)
        ms.append(m)
        js.append(j)
        if k < 2:
            d = jnp.where(cand == j, jnp.float32(jnp.inf), d)

    r1 = 1.0 / (ms[0] + 1e-8)
    r2 = 1.0 / (ms[1] + 1e-8)
    r3 = 1.0 / (ms[2] + 1e-8)
    norm = r1 + r2 + r3
    w1[0] = r1 / norm
    w2[0] = r2 / norm
    w3[0] = r3 / norm
    base = b * S
    i1[0] = js[0] + base
    i2[0] = js[1] + base
    i3[0] = js[2] + base


def _nn3(xyz1, x2t):
    grid = (B, N // NB1)
    out = pl.pallas_call(
        _nn3_body,
        grid=grid,
        in_specs=[
            pl.BlockSpec((1, 3, NB1), lambda b, n: (b, 0, n)),
            pl.BlockSpec((1, S, 3), lambda b, n: (b, 0, 0)),
        ],
        out_specs=[pl.BlockSpec((1, 1, NB1), lambda b, n: (b, 0, n))] * 6,
        out_shape=[jax.ShapeDtypeStruct((B, 1, N), jnp.int32)] * 3
        + [jax.ShapeDtypeStruct((B, 1, N), jnp.float32)] * 3,
    )(xyz1, x2t)
    return out


def _sc_interp_body(p2_hbm, i1_hbm, i2_hbm, i3_hbm, w1_hbm, w2_hbm, w3_hbm,
                    out_hbm, i1_v, i2_v, i3_v, w1_v, w2_v, w3_v,
                    r1_v, r2_v, r3_v, o_v, sem):
    wid = lax.axis_index("s") * NC + lax.axis_index("c")
    base = wid * RPW
    pltpu.sync_copy(i1_hbm.at[pl.ds(base, RPW)], i1_v)
    pltpu.sync_copy(i2_hbm.at[pl.ds(base, RPW)], i2_v)
    pltpu.sync_copy(i3_hbm.at[pl.ds(base, RPW)], i3_v)
    pltpu.sync_copy(w1_hbm.at[pl.ds(base, RPW)], w1_v)
    pltpu.sync_copy(w2_hbm.at[pl.ds(base, RPW)], w2_v)
    pltpu.sync_copy(w3_hbm.at[pl.ds(base, RPW)], w3_v)

    def round_body(g, _):
        off = g * CHUNK
        cp1 = pltpu.async_copy(p2_hbm.at[i1_v.at[pl.ds(off, CHUNK)]], r1_v, sem)
        cp2 = pltpu.async_copy(p2_hbm.at[i2_v.at[pl.ds(off, CHUNK)]], r2_v, sem)
        cp3 = pltpu.async_copy(p2_hbm.at[i3_v.at[pl.ds(off, CHUNK)]], r3_v, sem)
        cp1.wait()
        cp2.wait()
        cp3.wait()

        def sub_body(t, _):
            r0 = t * SUB
            wv1 = w1_v[pl.ds(off + r0, SUB)]
            wv2 = w2_v[pl.ds(off + r0, SUB)]
            wv3 = w3_v[pl.ds(off + r0, SUB)]
            for r in range(SUB):
                ids = jnp.full((LANES,), r, jnp.int32)
                a1 = wv1.at[ids].get(mode="promise_in_bounds")
                a2 = wv2.at[ids].get(mode="promise_in_bounds")
                a3 = wv3.at[ids].get(mode="promise_in_bounds")
                row = r0 + r
                for c in range(D2 // LANES):
                    sl = pl.ds(c * LANES, LANES)
                    o_v[row, sl] = (a1 * r1_v[row, sl]
                                    + a2 * r2_v[row, sl]
                                    + a3 * r3_v[row, sl])
            return _

        lax.fori_loop(0, CHUNK // SUB, sub_body, 0, unroll=False)
        pltpu.sync_copy(o_v, out_hbm.at[pl.ds(base + off, CHUNK)])
        return _

    lax.fori_loop(0, RPW // CHUNK, round_body, 0, unroll=False)


def _sc_interp(p2_flat, i1, i2, i3, w1, w2, w3):
    mesh = plsc.VectorSubcoreMesh(core_axis_name="c", subcore_axis_name="s")
    fn = functools.partial(
        pl.kernel,
        out_type=jax.ShapeDtypeStruct((R, D2), jnp.float32),
        mesh=mesh,
        scratch_types=[
            pltpu.VMEM((RPW,), jnp.int32),
            pltpu.VMEM((RPW,), jnp.int32),
            pltpu.VMEM((RPW,), jnp.int32),
            pltpu.VMEM((RPW,), jnp.float32),
            pltpu.VMEM((RPW,), jnp.float32),
            pltpu.VMEM((RPW,), jnp.float32),
            pltpu.VMEM((CHUNK, D2), jnp.float32),
            pltpu.VMEM((CHUNK, D2), jnp.float32),
            pltpu.VMEM((CHUNK, D2), jnp.float32),
            pltpu.VMEM((CHUNK, D2), jnp.float32),
            pltpu.SemaphoreType.DMA,
        ],
    )(_sc_interp_body)
    return fn(p2_flat, i1, i2, i3, w1, w2, w3)


def _conv0_body(p1_ref, it_ref, w0_ref, b0_ref, y_ref, st_ref):
    first = (pl.program_id(0) == 0) & (pl.program_id(1) == 0)

    @pl.when(first)
    def _():
        st_ref[...] = jnp.zeros_like(st_ref)

    w0a = w0_ref[:, :D1]
    w0b = w0_ref[:, D1:]
    t1 = lax.dot_general(w0a, p1_ref[0], (((1,), (0,)), ((), ())),
                         preferred_element_type=jnp.float32)
    t2 = lax.dot_general(w0b, it_ref[0], (((1,), (1,)), ((), ())),
                         preferred_element_type=jnp.float32)
    y = t1 + t2 + b0_ref[...]
    y_ref[0] = y
    st_ref[:, 0:1] += jnp.sum(y, axis=1, keepdims=True)
    st_ref[:, 1:2] += jnp.sum(y * y, axis=1, keepdims=True)


def _conv0(points1, interp3d, W0, b0c):
    grid = (B, N // NB3)
    return pl.pallas_call(
        _conv0_body,
        grid=grid,
        in_specs=[
            pl.BlockSpec((1, D1, NB3), lambda b, n: (b, 0, n)),
            pl.BlockSpec((1, NB3, D2), lambda b, n: (b, n, 0)),
            pl.BlockSpec((MLP0, D1 + D2), lambda b, n: (0, 0)),
            pl.BlockSpec((MLP0, 1), lambda b, n: (0, 0)),
        ],
        out_specs=[
            pl.BlockSpec((1, MLP0, NB3), lambda b, n: (b, 0, n)),
            pl.BlockSpec((MLP0, 128), lambda b, n: (0, 0)),
        ],
        out_shape=[
            jax.ShapeDtypeStruct((B, MLP0, N), jnp.float32),
            jax.ShapeDtypeStruct((MLP0, 128), jnp.float32),
        ],
    )(points1, interp3d, W0, b0c)


def _conv1_body(y0_ref, st0_ref, g0_ref, be0_ref, w1_ref, b1_ref,
                y_ref, st_ref):
    first = (pl.program_id(0) == 0) & (pl.program_id(1) == 0)

    @pl.when(first)
    def _():
        st_ref[...] = jnp.zeros_like(st_ref)

    cnt = jnp.float32(B * N)
    mean = st0_ref[:, 0:1] / cnt
    var = st0_ref[:, 1:2] / cnt - mean * mean
    inv = lax.rsqrt(var + 1e-5)
    scale = g0_ref[...] * inv
    shift = be0_ref[...] - mean * scale
    h = jnp.maximum(y0_ref[0] * scale + shift, 0.0)
    y = lax.dot_general(w1_ref[...], h, (((1,), (0,)), ((), ())),
                        preferred_element_type=jnp.float32) + b1_ref[...]
    y_ref[0] = y
    st_ref[:, 0:1] += jnp.sum(y, axis=1, keepdims=True)
    st_ref[:, 1:2] += jnp.sum(y * y, axis=1, keepdims=True)


def _conv1(y0, st0, g0c, beta0c, W1, b1c):
    grid = (B, N // NB4)
    return pl.pallas_call(
        _conv1_body,
        grid=grid,
        in_specs=[
            pl.BlockSpec((1, MLP0, NB4), lambda b, n: (b, 0, n)),
            pl.BlockSpec((MLP0, 128), lambda b, n: (0, 0)),
            pl.BlockSpec((MLP0, 1), lambda b, n: (0, 0)),
            pl.BlockSpec((MLP0, 1), lambda b, n: (0, 0)),
            pl.BlockSpec((MLP1, MLP0), lambda b, n: (0, 0)),
            pl.BlockSpec((MLP1, 1), lambda b, n: (0, 0)),
        ],
        out_specs=[
            pl.BlockSpec((1, MLP1, NB4), lambda b, n: (b, 0, n)),
            pl.BlockSpec((MLP1, 128), lambda b, n: (0, 0)),
        ],
        out_shape=[
            jax.ShapeDtypeStruct((B, MLP1, N), jnp.float32),
            jax.ShapeDtypeStruct((MLP1, 128), jnp.float32),
        ],
    )(y0, st0, g0c, beta0c, W1, b1c)


def _bn1_body(y1_ref, st1_ref, g1_ref, be1_ref, out_ref):
    cnt = jnp.float32(B * N)
    mean = st1_ref[:, 0:1] / cnt
    var = st1_ref[:, 1:2] / cnt - mean * mean
    inv = lax.rsqrt(var + 1e-5)
    scale = g1_ref[...] * inv
    shift = be1_ref[...] - mean * scale
    out_ref[0] = jnp.maximum(y1_ref[0] * scale + shift, 0.0)


def _bn1(y1, st1, g1c, beta1c):
    grid = (B, N // NB5)
    return pl.pallas_call(
        _bn1_body,
        grid=grid,
        in_specs=[
            pl.BlockSpec((1, MLP1, NB5), lambda b, n: (b, 0, n)),
            pl.BlockSpec((MLP1, 128), lambda b, n: (0, 0)),
            pl.BlockSpec((MLP1, 1), lambda b, n: (0, 0)),
            pl.BlockSpec((MLP1, 1), lambda b, n: (0, 0)),
        ],
        out_specs=pl.BlockSpec((1, MLP1, NB5), lambda b, n: (b, 0, n)),
        out_shape=jax.ShapeDtypeStruct((B, MLP1, N), jnp.float32),
    )(y1, st1, g1c, beta1c)


@jax.jit
def kernel(xyz1, xyz2, points1, points2, W0, b0, g0, beta0, W1, b1, g1, beta1):
    i1, i2, i3, w1, w2, w3 = _nn3(xyz1, jnp.transpose(xyz2, (0, 2, 1)))
    p2_flat = jnp.transpose(points2, (0, 2, 1)).reshape(B * S, D2)
    interp = _sc_interp(
        p2_flat,
        i1.reshape(R), i2.reshape(R), i3.reshape(R),
        w1.reshape(R), w2.reshape(R), w3.reshape(R),
    )
    interp3d = interp.reshape(B, N, D2)
    y0, st0 = _conv0(points1, interp3d, W0, b0.reshape(MLP0, 1))
    y1, st1 = _conv1(y0, st0, g0.reshape(MLP0, 1), beta0.reshape(MLP0, 1),
                     W1, b1.reshape(MLP1, 1))
    return _bn1(y1, st1, g1.reshape(MLP1, 1), beta1.reshape(MLP1, 1))
